# baseline (speedup 1.0000x reference)
import jax
import jax.numpy as jnp
from jax.experimental import pallas as pl
from jax.experimental.pallas import tpu as pltpu

H, D, K = 4096, 256, 8192
BETA = 0.25
N = 2 * 2048
BM = 256
NBLK = N // BM
KC = 2048
NKC = K // KC


def _vq_fused_kernel(emb_ref, wpre_ref, bpre_ref, cb2_ref, e2_ref,
                     prior_ref, wpost_ref, bpost_ref,
                     out_ref, idx_ref, part_ref):
    z = jax.lax.dot_general(
        emb_ref[...], wpre_ref[...],
        dimension_numbers=(((1,), (1,)), ((), ())),
        preferred_element_type=jnp.float32)
    z = z + bpre_ref[...]
    x2 = jnp.sum(z ** 2, axis=1, keepdims=True)

    fiota = jax.lax.broadcasted_iota(jnp.int32, (BM, KC), 1).astype(jnp.float32)
    m_run = jnp.full((BM, 1), jnp.inf, jnp.float32)
    idxf_run = jnp.zeros((BM, 1), jnp.float32)
    for c in range(NKC):
        cb_c = cb2_ref[pl.ds(c * KC, KC), :]
        xe2 = jax.lax.dot_general(
            z, cb_c,
            dimension_numbers=(((1,), (1,)), ((), ())),
            preferred_element_type=jnp.float32)
        dist = (x2 + e2_ref[:, pl.ds(c * KC, KC)]) - xe2
        m_c = jnp.min(dist, axis=1, keepdims=True)
        idxf_c = jnp.min(
            jnp.where(dist == m_c, fiota, jnp.float32(K)),
            axis=1, keepdims=True)
        better = m_c < m_run
        idxf_run = jnp.where(better, idxf_c + jnp.float32(c * KC), idxf_run)
        m_run = jnp.minimum(m_run, m_c)

    idx_ref[0, :, :] = idxf_run.astype(jnp.int32)

    xq2 = jnp.zeros((BM, D), jnp.float32)
    plog = jnp.zeros((BM, 1), jnp.float32)
    for c in range(NKC):
        onehot = jnp.where(fiota == idxf_run - jnp.float32(c * KC),
                           jnp.float32(1.0), jnp.float32(0.0))
        xq2 = xq2 + jax.lax.dot_general(
            onehot, cb2_ref[pl.ds(c * KC, KC), :],
            dimension_numbers=(((1,), (0,)), ((), ())),
            preferred_element_type=jnp.float32)
        plog = plog + jax.lax.dot_general(
            onehot, prior_ref[:, pl.ds(c * KC, KC)],
            dimension_numbers=(((1,), (1,)), ((), ())),
            preferred_element_type=jnp.float32)
    x_q = 0.5 * xq2

    x_q_st = z + (x_q - z)
    out = jax.lax.dot_general(
        x_q_st, wpost_ref[...],
        dimension_numbers=(((1,), (1,)), ((), ())),
        preferred_element_type=jnp.float32)
    out_ref[...] = out + bpost_ref[...]

    diff = x_q - z
    sum_sq = jnp.sum(diff * diff)
    sum_plog = jnp.sum(plog)
    lane = jax.lax.broadcasted_iota(jnp.int32, (1, 128), 1)
    part = jnp.where(lane == 0, sum_sq, jnp.where(lane == 1, sum_plog, 0.0))
    part_ref[0, ...] = part


def kernel(embed, W_pre, b_pre, codebook, W_post, b_post, prior_logits):
    emb2d = embed.reshape(N, H)
    e2 = jnp.sum(codebook ** 2, axis=1)[None, :]
    cb2 = codebook * 2.0

    embed_hat2d, idx3, parts = pl.pallas_call(
        _vq_fused_kernel,
        grid=(NBLK,),
        compiler_params=pltpu.CompilerParams(
            dimension_semantics=("parallel",)),
        in_specs=[
            pl.BlockSpec((BM, H), lambda i: (i, 0)),
            pl.BlockSpec((D, H), lambda i: (0, 0)),
            pl.BlockSpec((1, D), lambda i: (0, 0)),
            pl.BlockSpec((K, D), lambda i: (0, 0)),
            pl.BlockSpec((1, K), lambda i: (0, 0)),
            pl.BlockSpec((1, K), lambda i: (0, 0)),
            pl.BlockSpec((H, D), lambda i: (0, 0)),
            pl.BlockSpec((1, H), lambda i: (0, 0)),
        ],
        out_specs=[
            pl.BlockSpec((BM, H), lambda i: (i, 0)),
            pl.BlockSpec((1, BM, 1), lambda i: (i, 0, 0)),
            pl.BlockSpec((1, 1, 128), lambda i: (i, 0, 0)),
        ],
        out_shape=[
            jax.ShapeDtypeStruct((N, H), jnp.float32),
            jax.ShapeDtypeStruct((NBLK, BM, 1), jnp.int32),
            jax.ShapeDtypeStruct((NBLK, 1, 128), jnp.float32),
        ],
    )(emb2d, W_pre, b_pre.reshape(1, D), cb2, e2,
      prior_logits.reshape(1, K), W_post, b_post.reshape(1, H))

    embed_hat = embed_hat2d.reshape(embed.shape)
    idx = idx3.reshape(N)
    sum_sq = jnp.sum(parts[:, 0, 0])
    sum_plog = jnp.sum(parts[:, 0, 1])
    mean_sq = sum_sq / (N * D)
    vq_loss = mean_sq + BETA * mean_sq
    lse = jax.nn.logsumexp(prior_logits)
    rate_bits = (N * lse - sum_plog) / jnp.log(2.0)
    return (embed_hat, idx, rate_bits, vq_loss)


# speedup vs baseline: 1.0170x; 1.0170x over previous
import jax
import jax.numpy as jnp
from jax.experimental import pallas as pl
from jax.experimental.pallas import tpu as pltpu

H, D, K = 4096, 256, 8192
BETA = 0.25
N = 2 * 2048
BM = 256
NBLK = N // BM
KC = 2048
NKC = K // KC
GW = D + 128


def _vq_fused_kernel(emb_ref, wpre_ref, bpre_ref, g_ref, e2_ref,
                     wpost_ref, bpost_ref,
                     out_ref, idx_ref, part_ref):
    z = jax.lax.dot_general(
        emb_ref[...], wpre_ref[...],
        dimension_numbers=(((1,), (1,)), ((), ())),
        preferred_element_type=jnp.float32)
    z = z + bpre_ref[...]
    x2 = jnp.sum(z ** 2, axis=1, keepdims=True)

    fiota = jax.lax.broadcasted_iota(jnp.int32, (BM, KC), 1).astype(jnp.float32)
    mvec = jnp.full((BM, KC), jnp.inf, jnp.float32)
    ckvec = jnp.zeros((BM, KC), jnp.float32)
    for c in range(NKC):
        xe2 = jax.lax.dot_general(
            z, g_ref[pl.ds(c * KC, KC), :D],
            dimension_numbers=(((1,), (1,)), ((), ())),
            preferred_element_type=jnp.float32)
        dist = (x2 + e2_ref[:, pl.ds(c * KC, KC)]) - xe2
        upd = dist < mvec
        mvec = jnp.where(upd, dist, mvec)
        ckvec = jnp.where(upd, jnp.float32(c), ckvec)

    m = jnp.min(mvec, axis=1, keepdims=True)
    fidx = ckvec * jnp.float32(KC) + fiota
    idxf = jnp.min(jnp.where(mvec == m, fidx, jnp.float32(K)),
                   axis=1, keepdims=True)
    idx_ref[0, :, :] = idxf.astype(jnp.int32)

    acc = jnp.zeros((BM, GW), jnp.float32)
    for c in range(NKC):
        onehot = jnp.where(fiota == idxf - jnp.float32(c * KC),
                           jnp.float32(1.0), jnp.float32(0.0))
        acc = acc + jax.lax.dot_general(
            onehot, g_ref[pl.ds(c * KC, KC), :],
            dimension_numbers=(((1,), (0,)), ((), ())),
            preferred_element_type=jnp.float32)
    x_q = 0.5 * acc[:, :D]

    x_q_st = z + (x_q - z)
    out = jax.lax.dot_general(
        x_q_st, wpost_ref[...],
        dimension_numbers=(((1,), (1,)), ((), ())),
        preferred_element_type=jnp.float32)
    out_ref[...] = out + bpost_ref[...]

    diff = x_q - z
    sum_sq = jnp.sum(diff * diff)
    sum_plog = 0.5 * jnp.sum(acc[:, D:D + 1])
    lane = jax.lax.broadcasted_iota(jnp.int32, (1, 128), 1)
    part = jnp.where(lane == 0, sum_sq, jnp.where(lane == 1, sum_plog, 0.0))
    part_ref[0, ...] = part


def kernel(embed, W_pre, b_pre, codebook, W_post, b_post, prior_logits):
    emb2d = embed.reshape(N, H)
    e2 = jnp.sum(codebook ** 2, axis=1)[None, :]
    gmat = jnp.concatenate(
        [codebook * 2.0,
         prior_logits[:, None] * 2.0,
         jnp.zeros((K, GW - D - 1), jnp.float32)], axis=1)

    embed_hat2d, idx3, parts = pl.pallas_call(
        _vq_fused_kernel,
        grid=(NBLK,),
        compiler_params=pltpu.CompilerParams(
            dimension_semantics=("parallel",)),
        in_specs=[
            pl.BlockSpec((BM, H), lambda i: (i, 0)),
            pl.BlockSpec((D, H), lambda i: (0, 0)),
            pl.BlockSpec((1, D), lambda i: (0, 0)),
            pl.BlockSpec((K, GW), lambda i: (0, 0)),
            pl.BlockSpec((1, K), lambda i: (0, 0)),
            pl.BlockSpec((H, D), lambda i: (0, 0)),
            pl.BlockSpec((1, H), lambda i: (0, 0)),
        ],
        out_specs=[
            pl.BlockSpec((BM, H), lambda i: (i, 0)),
            pl.BlockSpec((1, BM, 1), lambda i: (i, 0, 0)),
            pl.BlockSpec((1, 1, 128), lambda i: (i, 0, 0)),
        ],
        out_shape=[
            jax.ShapeDtypeStruct((N, H), jnp.float32),
            jax.ShapeDtypeStruct((NBLK, BM, 1), jnp.int32),
            jax.ShapeDtypeStruct((NBLK, 1, 128), jnp.float32),
        ],
    )(emb2d, W_pre, b_pre.reshape(1, D), gmat, e2,
      W_post, b_post.reshape(1, H))

    embed_hat = embed_hat2d.reshape(embed.shape)
    idx = idx3.reshape(N)
    sum_sq = jnp.sum(parts[:, 0, 0])
    sum_plog = jnp.sum(parts[:, 0, 1])
    mean_sq = sum_sq / (N * D)
    vq_loss = mean_sq + BETA * mean_sq
    lse = jax.nn.logsumexp(prior_logits)
    rate_bits = (N * lse - sum_plog) / jnp.log(2.0)
    return (embed_hat, idx, rate_bits, vq_loss)
